# Initial kernel scaffold; baseline (speedup 1.0000x reference)
import jax
import jax.numpy as jnp
from jax.experimental import pallas as pl


def kernel(x_0, k, q, v, t, neighbors, times, w_t2v, b_t2v, w_tp, b_tp, w_proj, b_proj):
    raise NotImplementedError("write your pallas kernel here")


# speedup vs baseline: 1.4659x; 1.4659x over previous
import jax
import jax.numpy as jnp
from jax.experimental import pallas as pl
from jax.experimental.pallas import tpu as pltpu

_D = 128
_HALF = 64
_QROWS = 8


def _fused_kernel(x0_ref, q_blk_ref, t_ref, times_ref, w_t2v_ref, b_t2v_ref,
                  wtp_a_ref, wtp_b_ref, b_tp_ref, w_proj_ref, b_proj_ref,
                  out_ref):
    t = t_ref[0, 0]
    tmax = jnp.maximum(jnp.max(times_ref[:]), t)
    delta = tmax - t
    s = delta * w_t2v_ref[:] + b_t2v_ref[:]
    te = (jnp.dot(jnp.sin(s), wtp_a_ref[:],
                  preferred_element_type=jnp.float32)
          + jnp.dot(jnp.cos(s), wtp_b_ref[:],
                    preferred_element_type=jnp.float32))
    te = te * jnp.sqrt(jnp.float32(_HALF)) + b_tp_ref[:]
    row = x0_ref[0] % _QROWS
    q0 = q_blk_ref[pl.ds(row, 1), :] + te
    out = jnp.dot(q0, w_proj_ref[:], preferred_element_type=jnp.float32)
    out_ref[:] = jnp.maximum(out + b_proj_ref[:], 0.0)


def kernel(x_0, k, q, v, t, neighbors, times, w_t2v, b_t2v, w_tp, b_tp,
           w_proj, b_proj):
    x0 = jnp.asarray(x_0, jnp.int32).reshape(1)
    t_f = jnp.asarray(t, jnp.float32).reshape(1, 1)
    times_row = jnp.asarray(times, jnp.float32).reshape(1, -1)
    b_t2v_row = b_t2v.reshape(1, _HALF)
    b_tp_row = b_tp.reshape(1, _D)
    b_proj_row = b_proj.reshape(1, _D)
    wtp_a = w_tp[:_HALF]
    wtp_b = w_tp[_HALF:]

    full = lambda arr: pl.BlockSpec(arr.shape, lambda i, x0r: (0, 0))
    grid_spec = pltpu.PrefetchScalarGridSpec(
        num_scalar_prefetch=1,
        grid=(1,),
        in_specs=[
            pl.BlockSpec((_QROWS, _D), lambda i, x0r: (x0r[0] // _QROWS, 0)),
            full(t_f),
            full(times_row),
            full(w_t2v),
            full(b_t2v_row),
            full(wtp_a),
            full(wtp_b),
            full(b_tp_row),
            full(w_proj),
            full(b_proj_row),
        ],
        out_specs=pl.BlockSpec((1, _D), lambda i, x0r: (0, 0)),
    )
    return pl.pallas_call(
        _fused_kernel,
        grid_spec=grid_spec,
        out_shape=jax.ShapeDtypeStruct((1, _D), jnp.float32),
    )(x0, q, t_f, times_row, w_t2v, b_t2v_row, wtp_a, wtp_b, b_tp_row,
      w_proj, b_proj_row)


# speedup vs baseline: 1.4818x; 1.0109x over previous
import jax
import jax.numpy as jnp
from jax.experimental import pallas as pl
from jax.experimental.pallas import tpu as pltpu

_D = 128
_HALF = 64
_QROWS = 8


def _fused_kernel(x0_ref, t_ref, q_blk_ref, times_ref, w_t2v_ref, b_t2v_ref,
                  w_tp_ref, b_tp_ref, w_proj_ref, b_proj_ref, out_ref):
    t = t_ref[0].astype(jnp.float32)
    tmax = jnp.maximum(jnp.max(times_ref[:]), t)
    delta = tmax - t
    s = delta * w_t2v_ref[:] + b_t2v_ref[:]
    emb = jnp.concatenate([jnp.sin(s), jnp.cos(s)], axis=1)
    emb = emb * jnp.sqrt(jnp.float32(_HALF))
    te = jnp.dot(emb, w_tp_ref[:], preferred_element_type=jnp.float32)
    te = te + b_tp_ref[:]
    row = x0_ref[0] % _QROWS
    q0 = q_blk_ref[pl.ds(row, 1), :] + te
    out = jnp.dot(q0, w_proj_ref[:], preferred_element_type=jnp.float32)
    out_ref[:] = jnp.maximum(out + b_proj_ref[:], 0.0)


def kernel(x_0, k, q, v, t, neighbors, times, w_t2v, b_t2v, w_tp, b_tp,
           w_proj, b_proj):
    x0 = jnp.asarray(x_0, jnp.int32).reshape(1)
    t_arr = jnp.asarray(t, jnp.int32).reshape(1)
    b_t2v_row = b_t2v.reshape(1, _HALF)
    b_tp_row = b_tp.reshape(1, _D)
    b_proj_row = b_proj.reshape(1, _D)

    full = lambda arr: pl.BlockSpec(arr.shape, lambda i, x0r, tr: (0, 0))
    grid_spec = pltpu.PrefetchScalarGridSpec(
        num_scalar_prefetch=2,
        grid=(1,),
        in_specs=[
            pl.BlockSpec((_QROWS, _D),
                         lambda i, x0r, tr: (x0r[0] // _QROWS, 0)),
            full(times),
            full(w_t2v),
            full(b_t2v_row),
            full(w_tp),
            full(b_tp_row),
            full(w_proj),
            full(b_proj_row),
        ],
        out_specs=pl.BlockSpec((1, _D), lambda i, x0r, tr: (0, 0)),
    )
    return pl.pallas_call(
        _fused_kernel,
        grid_spec=grid_spec,
        out_shape=jax.ShapeDtypeStruct((1, _D), jnp.float32),
    )(x0, t_arr, q, times, w_t2v, b_t2v_row, w_tp, b_tp_row,
      w_proj, b_proj_row)
